# baseline (speedup 1.0000x reference)
import functools

import jax
import jax.numpy as jnp
from jax.experimental import pallas as pl

NUM_CLASSES_K = 100000
SMOOTHING_K = 0.1
CONFIDENCE_K = 1.0 - SMOOTHING_K
EPS_K = SMOOTHING_K / (NUM_CLASSES_K - 1)

ROWS = 1024
ROW_BLOCK = 16


def _loss_kernel(pred_ref, tgt_ref, out_ref, *, num_blocks, num_cols,
                 total_rows):
    i = pl.program_id(0)

    x = pred_ref[...]
    m = jnp.max(x, axis=1, keepdims=True)
    s = jnp.sum(jnp.exp(x - m), axis=1, keepdims=True)
    t = jnp.sum(x, axis=1, keepdims=True)

    col = jax.lax.broadcasted_iota(jnp.int32, x.shape, 1)
    hit = col == tgt_ref[...]
    g = jnp.sum(jnp.where(hit, x, 0.0), axis=1, keepdims=True)

    lse = m + jnp.log(s)
    row_loss = (-EPS_K * (t - num_cols * lse)
                - (CONFIDENCE_K - EPS_K) * (g - lse))
    partial = jnp.sum(row_loss).reshape(1, 1) / total_rows

    @pl.when(i == 0)
    def _init():
        out_ref[...] = jnp.zeros_like(out_ref)

    out_ref[...] += partial


def kernel(pred, target):
    rows, num_cols = pred.shape
    num_blocks = rows // ROW_BLOCK
    tgt2d = target.astype(jnp.int32).reshape(rows, 1)

    out = pl.pallas_call(
        functools.partial(_loss_kernel, num_blocks=num_blocks,
                          num_cols=num_cols, total_rows=rows),
        grid=(num_blocks,),
        in_specs=[
            pl.BlockSpec((ROW_BLOCK, num_cols), lambda i: (i, 0)),
            pl.BlockSpec((ROW_BLOCK, 1), lambda i: (i, 0)),
        ],
        out_specs=pl.BlockSpec((1, 1), lambda i: (0, 0)),
        out_shape=jax.ShapeDtypeStruct((1, 1), jnp.float32),
    )(pred, tgt2d)
    return out[0, 0]
